# Initial kernel scaffold; baseline (speedup 1.0000x reference)
import jax
import jax.numpy as jnp
from jax.experimental import pallas as pl


def kernel(node_type, node_chiral_type, edge_type, edge_dire_type, edge_index, node_emb1, node_emb2, W1, b1, W2, b2, eemb1, eemb2, bn_g, bn_b):
    raise NotImplementedError("write your pallas kernel here")


# speedup vs baseline: 3.4868x; 3.4868x over previous
import functools

import jax
import jax.numpy as jnp
from jax import lax
from jax.experimental import pallas as pl
from jax.experimental.pallas import tpu as pltpu
from jax.experimental.pallas import tpu_sc as plsc

N = 10000
E = 160000
D = 256
H = 512
HALF = 128
NL = 5

RB = 1000
NB = N // RB
NSC = 2
NT = 16
NPAD = 10240
ROWS_PER_TILE = NPAD // NT
EDGES_PER_TILE = E // NT
EK = 200


def _mesh():
    return plsc.VectorSubcoreMesh(
        core_axis_name="c", subcore_axis_name="s",
        num_cores=NSC, num_subcores=NT)


@functools.cache
def _make_sc_x():
  @functools.partial(
      pl.kernel,
      out_type=jax.ShapeDtypeStruct((NSC, NPAD, HALF), jnp.float32),
      mesh=_mesh(),
      scratch_types=[
          pltpu.VMEM((ROWS_PER_TILE,), jnp.int32),
          pltpu.VMEM((ROWS_PER_TILE,), jnp.int32),
          pltpu.VMEM((ROWS_PER_TILE, HALF), jnp.float32),
          pltpu.SemaphoreType.DMA,
      ],
  )
  def _sc_x_k(emb1_hbm, emb2_hbm, ids1_hbm, ids2_hbm, out_hbm,
              idx1, idx2, rows, sem):
    c = lax.axis_index("c")
    s = lax.axis_index("s")
    rbase = s * ROWS_PER_TILE
    pltpu.sync_copy(ids1_hbm.at[pl.ds(rbase, ROWS_PER_TILE)], idx1)
    pltpu.sync_copy(ids2_hbm.at[pl.ds(rbase, ROWS_PER_TILE)], idx2)
    pltpu.async_copy(emb1_hbm.at[c].at[idx1], rows, sem).wait()
    pltpu.async_copy(emb2_hbm.at[c].at[idx2], rows, sem, add=True).wait()
    pltpu.sync_copy(rows, out_hbm.at[c, pl.ds(rbase, ROWS_PER_TILE)])

  return _sc_x_k


def _sc_x(emb1, emb2, ids1, ids2):
    return _make_sc_x()(emb1, emb2, ids1, ids2)


@functools.cache
def _make_sc_aggr():
  @functools.partial(
      pl.kernel,
      out_type=jax.ShapeDtypeStruct((NSC, NPAD, HALF), jnp.float32),
      mesh=_mesh(),
      scratch_types=[
          pltpu.VMEM_SHARED((NPAD, HALF), jnp.float32),
          pltpu.VMEM((EK,), jnp.int32),
          pltpu.VMEM((EK,), jnp.int32),
          pltpu.VMEM((EK,), jnp.int32),
          pltpu.VMEM((EK, HALF), jnp.float32),
          pltpu.SemaphoreType.DMA,
      ],
  )
  def _sc_aggr_k(h_hbm, tab_hbm, src_hbm, dst_hbm, combo_hbm, zero_hbm,
                 out_hbm, acc, sidx, kidx, didx, rows, sem):
    c = lax.axis_index("c")
    s = lax.axis_index("s")
    rbase = s * ROWS_PER_TILE
    pltpu.sync_copy(zero_hbm.at[pl.ds(rbase, ROWS_PER_TILE)],
                    acc.at[pl.ds(rbase, ROWS_PER_TILE)])
    plsc.subcore_barrier()

    ebase = s * EDGES_PER_TILE

    def body(j, carry):
        off = pl.multiple_of(ebase + j * EK, 8)
        pltpu.sync_copy(src_hbm.at[pl.ds(off, EK)], sidx)
        pltpu.sync_copy(combo_hbm.at[pl.ds(off, EK)], kidx)
        pltpu.async_copy(h_hbm.at[c].at[sidx], rows, sem).wait()
        pltpu.async_copy(tab_hbm.at[c].at[kidx], rows, sem, add=True).wait()
        pltpu.sync_copy(dst_hbm.at[pl.ds(off, EK)], didx)
        pltpu.sync_copy(rows, acc.at[didx], add=True)
        return carry

    lax.fori_loop(0, EDGES_PER_TILE // EK, body, 0)
    plsc.subcore_barrier()
    pltpu.sync_copy(acc.at[pl.ds(rbase, ROWS_PER_TILE)],
                    out_hbm.at[c, pl.ds(rbase, ROWS_PER_TILE)])

  return _sc_aggr_k


def _sc_aggr(h, tab, src, dst, combo, zeros):
    return _make_sc_aggr()(h, tab, src, dst, combo, zeros)


_DEF = lax.Precision.DEFAULT


def _mlp_body(ag_ref, h_ref, row_ref, w1_ref, b1_ref, w2_ref, b2_ref,
              hnew_ref):
    self0 = h_ref[0] + row_ref[0:1, :HALF]
    self1 = h_ref[1] + row_ref[0:1, HALF:]
    z = jnp.concatenate([ag_ref[0] + self0, ag_ref[1] + self1], axis=1)
    hmid = jnp.dot(z, w1_ref[...], preferred_element_type=jnp.float32,
                   precision=_DEF)
    hmid = jnp.maximum(hmid + b1_ref[...], 0.0)
    hnew = jnp.dot(hmid, w2_ref[...], preferred_element_type=jnp.float32,
                   precision=_DEF)
    hnew_ref[...] = hnew + b2_ref[...]


def _mlp_call(ag, h, row, w1, b1, w2, b2):
    return pl.pallas_call(
        _mlp_body,
        grid=(NB,),
        in_specs=[
            pl.BlockSpec((NSC, RB, HALF), lambda b: (0, b, 0)),
            pl.BlockSpec((NSC, RB, HALF), lambda b: (0, b, 0)),
            pl.BlockSpec((1, D), lambda b: (0, 0)),
            pl.BlockSpec((D, H), lambda b: (0, 0)),
            pl.BlockSpec((1, H), lambda b: (0, 0)),
            pl.BlockSpec((H, D), lambda b: (0, 0)),
            pl.BlockSpec((1, D), lambda b: (0, 0)),
        ],
        out_specs=pl.BlockSpec((RB, D), lambda b: (b, 0)),
        out_shape=jax.ShapeDtypeStruct((N, D), jnp.float32),
    )(ag, h, row, w1, b1, w2, b2)


def _bn_body_split(h_ref, mean_ref, var_ref, g_ref, bb_ref, out_ref):
    y = ((h_ref[...] - mean_ref[...]) / jnp.sqrt(var_ref[...] + 1e-5)
         * g_ref[...] + bb_ref[...])
    y = jnp.maximum(y, 0.0)
    out_ref[0] = y[:, :HALF]
    out_ref[1] = y[:, HALF:]


def _bn_body_final(h_ref, mean_ref, var_ref, g_ref, bb_ref, out_ref):
    out_ref[...] = ((h_ref[...] - mean_ref[...])
                    / jnp.sqrt(var_ref[...] + 1e-5)
                    * g_ref[...] + bb_ref[...])


def _bn_call(h, mean, var, g, bb, final):
    body = _bn_body_final if final else _bn_body_split
    if final:
        out_spec = pl.BlockSpec((RB, D), lambda b: (b, 0))
        out_shape = jax.ShapeDtypeStruct((N, D), jnp.float32)
    else:
        out_spec = pl.BlockSpec((NSC, RB, HALF), lambda b: (0, b, 0))
        out_shape = jax.ShapeDtypeStruct((NSC, NPAD, HALF), jnp.float32)
    return pl.pallas_call(
        body,
        grid=(NB,),
        in_specs=[
            pl.BlockSpec((RB, D), lambda b: (b, 0)),
            pl.BlockSpec((1, D), lambda b: (0, 0)),
            pl.BlockSpec((1, D), lambda b: (0, 0)),
            pl.BlockSpec((1, D), lambda b: (0, 0)),
            pl.BlockSpec((1, D), lambda b: (0, 0)),
        ],
        out_specs=out_spec,
        out_shape=out_shape,
    )(h, mean, var, g, bb)


def _split2(a):
    return jnp.stack([a[:, :HALF], a[:, HALF:]], axis=0)


def kernel(node_type, node_chiral_type, edge_type, edge_dire_type, edge_index,
           node_emb1, node_emb2, W1, b1, W2, b2, eemb1, eemb2, bn_g, bn_b):
    f32 = jnp.float32
    i32 = jnp.int32
    src = edge_index[0].astype(i32)
    dst = edge_index[1].astype(i32)
    combo = edge_type.astype(i32) * 3 + edge_dire_type.astype(i32)

    order = jnp.argsort(dst, stable=True)
    srcs = src[order]
    dsts = dst[order]
    combos = combo[order]

    ids1 = jnp.pad(node_type.astype(i32), (0, NPAD - N))
    ids2 = jnp.pad(node_chiral_type.astype(i32), (0, NPAD - N))
    emb1s = _split2(node_emb1.astype(f32))
    emb2s = _split2(node_emb2.astype(f32))

    h = _sc_x(emb1s, emb2s, ids1, ids2)
    zeros = jnp.zeros((NPAD, HALF), f32)

    t18 = jnp.arange(18, dtype=i32) // 3
    d18 = jnp.arange(18, dtype=i32) % 3

    out = None
    for i in range(NL):
        tab = _split2(eemb1[i][t18] + eemb2[i][d18])
        selfrow = (eemb1[i][4] + eemb2[i][0]).reshape(1, D)
        ag = _sc_aggr(h, tab, srcs, dsts, combos, zeros)
        hnew = _mlp_call(ag, h, selfrow, W1[i], b1[i].reshape(1, H),
                         W2[i], b2[i].reshape(1, D))
        mean = jnp.mean(hnew, axis=0)
        var = jnp.var(hnew, axis=0)
        final = i == NL - 1
        res = _bn_call(hnew, mean.reshape(1, D), var.reshape(1, D),
                       bn_g[i].reshape(1, D), bn_b[i].reshape(1, D), final)
        if final:
            out = res
        else:
            h = res
    return out
